# baseline (speedup 1.0000x reference)
import functools

import jax
import jax.numpy as jnp
from jax import lax
from jax.experimental import pallas as pl
from jax.experimental.pallas import tpu as pltpu
from jax.experimental.pallas import tpu_sc as plsc

VOCAB = 1000000
D = 32
SCALE = D ** 0.5

NC = 2
NS = 16
NW = NC * NS

N_BATCH = 16384
N_SLOT = 50
C = 128
N_PER_W = N_BATCH // NW
CPS = N_PER_W // C
CHUNKS = N_SLOT * CPS

TB = 64
NBLK = VOCAB // TB
BLK_BASE = NBLK // NW
BLK_REM = NBLK - BLK_BASE * NW


DNB = 8


def _detile_body(tt_hbm, out_hbm, *rest):
    in_b = rest[0:DNB]
    rm_b = rest[DNB:2 * DNB]
    s_in = rest[2 * DNB:3 * DNB]
    s_out = rest[3 * DNB:4 * DNB]

    wid = lax.axis_index("s") * NC + lax.axis_index("c")
    start = wid * BLK_BASE + jnp.minimum(wid, BLK_REM)
    count = BLK_BASE + (wid < BLK_REM).astype(jnp.int32)
    lanes = lax.iota(jnp.int32, 16)

    def issue_in(blk, p):
        n0 = blk * TB
        for k in range(4):
            pltpu.make_async_copy(
                tt_hbm.at[pl.ds(8 * k, 8), pl.ds(n0, TB)],
                in_b[p].at[pl.ds(8 * k, 8), pl.ds(0, TB)], s_in[p]).start()

    def wait_in(p):
        for k in range(4):
            pltpu.make_async_copy(
                tt_hbm.at[pl.ds(0, 8), pl.ds(0, TB)],
                in_b[p].at[pl.ds(8 * k, 8), pl.ds(0, TB)], s_in[p]).wait()

    def out_desc(blk, p):
        return pltpu.make_async_copy(
            rm_b[p], out_hbm.at[pl.ds(blk * TB * D, TB * D)], s_out[p])

    for p in range(DNB):
        @pl.when(p < count)
        def _prime():
            issue_in(start + p, p)

    def blk_fn(i, carry):
        for p in range(DNB):
            idx = i * DNB + p

            @pl.when(idx < count)
            def _do():
                blk = start + idx
                wait_in(p)

                @pl.when(idx >= DNB)
                def _wait_out():
                    out_desc(blk, p).wait()

                def g_fn(g, carry2):
                    perm = (lanes + g) & 15
                    for f0 in range(0, D, 16):
                        for bt in range(TB // 16):
                            tvec = bt * 16 + lanes
                            fvec = f0 + perm
                            v = plsc.load_gather(in_b[p], [fvec, tvec])
                            plsc.store_scatter(
                                rm_b[p], [tvec * D + fvec], v * SCALE)
                    return carry2

                lax.fori_loop(0, 16, g_fn, 0, unroll=4)

                out_desc(blk, p).start()

                @pl.when(idx + DNB < count)
                def _refill():
                    issue_in(blk + DNB, p)

        return carry

    lax.fori_loop(0, (BLK_BASE + DNB) // DNB, blk_fn, 0)

    for p in range(DNB):
        @pl.when(count >= DNB - p)
        def _drain():
            out_desc(start, p).wait()


def _body(idx_hbm, table_hbm, out_hbm, idx_v, *rest):
    rows_b = rest[0:CPS]
    trans_b = rest[CPS:2 * CPS]
    sem_in = rest[2 * CPS:3 * CPS]
    sem_out = rest[3 * CPS:4 * CPS]

    wid = lax.axis_index("s") * NC + lax.axis_index("c")
    n_base = wid * N_PER_W
    pltpu.sync_copy(idx_hbm.at[wid], idx_v)

    lanes = lax.iota(jnp.int32, 16)

    for c in range(CPS):
        pltpu.make_async_copy(
            table_hbm.at[idx_v.at[c]], rows_b[c], sem_in[c]).start()

    def slot_fn(s, carry):
        for c in range(CPS):
            j = s * CPS + c
            pltpu.make_async_copy(
                table_hbm.at[idx_v.at[j]], rows_b[c], sem_in[c]).wait()
            tcol = (n_base + c * C) // C
            out_desc = [
                pltpu.make_async_copy(
                    trans_b[c].at[pl.ds(8 * k, 8), pl.ds(0, C)],
                    out_hbm.at[4 * s + k, tcol], sem_out[c])
                for k in range(4)]

            @pl.when(s > 0)
            def _wait_prev():
                for k in range(4):
                    out_desc[k].wait()

            def row_fn(r, carry2):
                rcol = jnp.full((16,), 0, jnp.int32) + r
                for f0 in range(0, D, 16):
                    v = rows_b[c][r, pl.ds(f0, 16)]
                    plsc.store_scatter(trans_b[c], [lanes + f0, rcol], v)
                return carry2

            lax.fori_loop(0, C, row_fn, 0, unroll=8)

            @pl.when(s + 1 < N_SLOT)
            def _refill():
                jn = jnp.minimum(j + CPS, CHUNKS - 1)
                pltpu.make_async_copy(
                    table_hbm.at[idx_v.at[jn]], rows_b[c], sem_in[c]).start()

            for k in range(4):
                out_desc[k].start()
        return carry

    lax.fori_loop(0, N_SLOT, slot_fn, 0)

    for c in range(CPS):
        for k in range(4):
            pltpu.make_async_copy(
                trans_b[c].at[pl.ds(8 * k, 8), pl.ds(0, C)],
                out_hbm.at[4 * (N_SLOT - 1) + k, (n_base + c * C) // C],
                sem_out[c]).wait()


@functools.partial(jax.jit, static_argnums=())
def _detile(tt):
    mesh = plsc.VectorSubcoreMesh(core_axis_name="c", subcore_axis_name="s")
    scratch = [pltpu.VMEM((D, 2 * TB), jnp.float32) for _ in range(DNB)]
    scratch += [pltpu.VMEM((TB * D,), jnp.float32) for _ in range(DNB)]
    scratch += [pltpu.SemaphoreType.DMA for _ in range(2 * DNB)]
    k = pl.kernel(
        _detile_body,
        out_type=jax.ShapeDtypeStruct((VOCAB * D,), jnp.float32),
        mesh=mesh,
        scratch_types=scratch,
        compiler_params=pltpu.CompilerParams(
            use_tc_tiling_on_sc=True, needs_layout_passes=False),
    )
    return k(tt)


@functools.partial(jax.jit, static_argnums=())
def _lookup(idx, table):
    mesh = plsc.VectorSubcoreMesh(core_axis_name="c", subcore_axis_name="s")
    scratch = [pltpu.VMEM((CHUNKS, C), jnp.int32)]
    scratch += [pltpu.VMEM((C, D), jnp.float32) for _ in range(CPS)]
    scratch += [pltpu.VMEM((D, C + 1), jnp.float32) for _ in range(CPS)]
    scratch += [pltpu.SemaphoreType.DMA for _ in range(2 * CPS)]
    k = pl.kernel(
        _body,
        out_type=jax.ShapeDtypeStruct(
            (N_SLOT * D // 8, N_BATCH // C, 8, C), jnp.float32),
        mesh=mesh,
        scratch_types=scratch,
        compiler_params=pltpu.CompilerParams(
            use_tc_tiling_on_sc=False, needs_layout_passes=False),
    )
    return k(idx, table)


def kernel(inputs, embeddings):
    idx = (inputs.astype(jnp.int32).T
           .reshape(N_SLOT, NW, CPS, C)
           .transpose(1, 0, 2, 3)
           .reshape(NW, CHUNKS, C))
    table = _detile(embeddings.T).reshape(VOCAB, D)
    out = _lookup(idx, table)
    out = out.transpose(0, 2, 1, 3).reshape(N_SLOT, D, N_BATCH)
    return jnp.transpose(out, (2, 0, 1))


# speedup vs baseline: 1.3906x; 1.3906x over previous
import functools

import jax
import jax.numpy as jnp
from jax import lax
from jax.experimental import pallas as pl
from jax.experimental.pallas import tpu as pltpu
from jax.experimental.pallas import tpu_sc as plsc

VOCAB = 1000000
D = 32
SCALE = D ** 0.5

NC = 2
NS = 16
NW = NC * NS

N_BATCH = 16384
N_SLOT = 50
C = 128
N_PER_W = N_BATCH // NW
CPS = N_PER_W // C
CHUNKS = N_SLOT * CPS

TB = 128
NBLK = VOCAB // TB
TAIL = VOCAB - NBLK * TB
BLK_BASE = NBLK // NW
BLK_REM = NBLK - BLK_BASE * NW


DNB = 4


def _detile_body(tt_hbm, tail_hbm, out_hbm, *rest):
    in_b = rest[0:DNB]
    rm_b = rest[DNB:2 * DNB]
    s_in = rest[2 * DNB:3 * DNB]
    s_out = rest[3 * DNB:4 * DNB]

    wid = lax.axis_index("s") * NC + lax.axis_index("c")
    start = wid * BLK_BASE + jnp.minimum(wid, BLK_REM)
    count = BLK_BASE + (wid < BLK_REM).astype(jnp.int32)
    lanes = lax.iota(jnp.int32, 16)

    def issue_in(blk, p):
        n0 = blk * TB
        for k in range(4):
            pltpu.make_async_copy(
                tt_hbm.at[pl.ds(8 * k, 8), pl.ds(n0, TB)],
                in_b[p].at[pl.ds(8 * k, 8), pl.ds(0, TB)], s_in[p]).start()

    def wait_in(p):
        for k in range(4):
            pltpu.make_async_copy(
                tt_hbm.at[pl.ds(0, 8), pl.ds(0, TB)],
                in_b[p].at[pl.ds(8 * k, 8), pl.ds(0, TB)], s_in[p]).wait()

    def transpose_pass(p):
        def g_fn(g, carry2):
            perm = (lanes + g) & 15
            for f0 in range(0, D, 16):
                for bt in range(TB // 16):
                    tvec = bt * 16 + lanes
                    fvec = f0 + perm
                    v = plsc.load_gather(in_b[p], [fvec, tvec])
                    plsc.store_scatter(rm_b[p], [tvec * D + fvec], v * SCALE)
            return carry2
        return g_fn

    def out_desc(blk, p):
        return pltpu.make_async_copy(
            rm_b[p], out_hbm.at[pl.ds(blk * TB * D, TB * D)], s_out[p])

    for p in range(DNB):
        @pl.when(p < count)
        def _prime():
            issue_in(start + p, p)

    def blk_fn(i, carry):
        for p in range(DNB):
            idx = i * DNB + p

            @pl.when(idx < count)
            def _do():
                blk = start + idx
                wait_in(p)

                @pl.when(idx >= DNB)
                def _wait_out():
                    out_desc(blk, p).wait()

                lax.fori_loop(0, 16, transpose_pass(p), 0, unroll=2)

                out_desc(blk, p).start()

                @pl.when(idx + DNB < count)
                def _refill():
                    issue_in(blk + DNB, p)

        return carry

    lax.fori_loop(0, (BLK_BASE + DNB) // DNB, blk_fn, 0)

    for p in range(DNB):
        @pl.when(count >= DNB - p)
        def _drain():
            out_desc(start, p).wait()

    @pl.when(wid == NW - 1)
    def _tail():
        pltpu.sync_copy(tail_hbm, rm_b[0].at[pl.ds(0, TAIL * D)])
        pltpu.sync_copy(rm_b[0].at[pl.ds(0, TAIL * D)],
                        out_hbm.at[pl.ds(NBLK * TB * D, TAIL * D)])


def _body(idx_hbm, table_hbm, out_hbm, idx_v, *rest):
    rows_b = rest[0:CPS]
    trans_b = rest[CPS:2 * CPS]
    sem_in = rest[2 * CPS:3 * CPS]
    sem_out = rest[3 * CPS:4 * CPS]

    wid = lax.axis_index("s") * NC + lax.axis_index("c")
    n_base = wid * N_PER_W
    pltpu.sync_copy(idx_hbm.at[wid], idx_v)

    lanes = lax.iota(jnp.int32, 16)

    for c in range(CPS):
        pltpu.make_async_copy(
            table_hbm.at[idx_v.at[c]], rows_b[c], sem_in[c]).start()

    def slot_fn(s, carry):
        for c in range(CPS):
            j = s * CPS + c
            pltpu.make_async_copy(
                table_hbm.at[idx_v.at[j]], rows_b[c], sem_in[c]).wait()
            tcol = (n_base + c * C) // C
            out_desc = [
                pltpu.make_async_copy(
                    trans_b[c].at[pl.ds(8 * k, 8), pl.ds(0, C)],
                    out_hbm.at[4 * s + k, tcol], sem_out[c])
                for k in range(4)]

            @pl.when(s > 0)
            def _wait_prev():
                for k in range(4):
                    out_desc[k].wait()

            def row_fn(r, carry2):
                rcol = jnp.full((16,), 0, jnp.int32) + r
                for f0 in range(0, D, 16):
                    v = rows_b[c][r, pl.ds(f0, 16)]
                    plsc.store_scatter(trans_b[c], [lanes + f0, rcol], v)
                return carry2

            lax.fori_loop(0, C, row_fn, 0, unroll=8)

            @pl.when(s + 1 < N_SLOT)
            def _refill():
                jn = jnp.minimum(j + CPS, CHUNKS - 1)
                pltpu.make_async_copy(
                    table_hbm.at[idx_v.at[jn]], rows_b[c], sem_in[c]).start()

            for k in range(4):
                out_desc[k].start()
        return carry

    lax.fori_loop(0, N_SLOT, slot_fn, 0)

    for c in range(CPS):
        for k in range(4):
            pltpu.make_async_copy(
                trans_b[c].at[pl.ds(8 * k, 8), pl.ds(0, C)],
                out_hbm.at[4 * (N_SLOT - 1) + k, (n_base + c * C) // C],
                sem_out[c]).wait()


@functools.partial(jax.jit, static_argnums=())
def _detile(tt, tail):
    mesh = plsc.VectorSubcoreMesh(core_axis_name="c", subcore_axis_name="s")
    scratch = [pltpu.VMEM((D, 2 * TB), jnp.float32) for _ in range(DNB)]
    scratch += [pltpu.VMEM((TB * D,), jnp.float32) for _ in range(DNB)]
    scratch += [pltpu.SemaphoreType.DMA for _ in range(2 * DNB)]
    k = pl.kernel(
        _detile_body,
        out_type=jax.ShapeDtypeStruct((VOCAB * D,), jnp.float32),
        mesh=mesh,
        scratch_types=scratch,
        compiler_params=pltpu.CompilerParams(
            use_tc_tiling_on_sc=True, needs_layout_passes=False),
    )
    return k(tt, tail)


@functools.partial(jax.jit, static_argnums=())
def _lookup(idx, table):
    mesh = plsc.VectorSubcoreMesh(core_axis_name="c", subcore_axis_name="s")
    scratch = [pltpu.VMEM((CHUNKS, C), jnp.int32)]
    scratch += [pltpu.VMEM((C, D), jnp.float32) for _ in range(CPS)]
    scratch += [pltpu.VMEM((D, C + 1), jnp.float32) for _ in range(CPS)]
    scratch += [pltpu.SemaphoreType.DMA for _ in range(2 * CPS)]
    k = pl.kernel(
        _body,
        out_type=jax.ShapeDtypeStruct(
            (N_SLOT * D // 8, N_BATCH // C, 8, C), jnp.float32),
        mesh=mesh,
        scratch_types=scratch,
        compiler_params=pltpu.CompilerParams(
            use_tc_tiling_on_sc=False, needs_layout_passes=False),
    )
    return k(idx, table)


def kernel(inputs, embeddings):
    idx = (inputs.astype(jnp.int32).T
           .reshape(N_SLOT, NW, CPS, C)
           .transpose(1, 0, 2, 3)
           .reshape(NW, CHUNKS, C))
    tail = (embeddings[NBLK * TB:] * SCALE).reshape(TAIL * D)
    table = _detile(embeddings.T, tail).reshape(VOCAB, D)
    out = _lookup(idx, table)
    out = out.transpose(0, 2, 1, 3).reshape(N_SLOT, D, N_BATCH)
    return jnp.transpose(out, (2, 0, 1))


# speedup vs baseline: 1.4507x; 1.0433x over previous
import functools

import jax
import jax.numpy as jnp
from jax import lax
from jax.experimental import pallas as pl
from jax.experimental.pallas import tpu as pltpu
from jax.experimental.pallas import tpu_sc as plsc

VOCAB = 1000000
D = 32
SCALE = D ** 0.5

NC = 2
NS = 16
NW = NC * NS

N_BATCH = 16384
N_SLOT = 50
C = 128
N_PER_W = N_BATCH // NW
CPS = N_PER_W // C
CHUNKS = N_SLOT * CPS

TB = 64
NBLK = VOCAB // TB
BLK_BASE = NBLK // NW
BLK_REM = NBLK - BLK_BASE * NW


DNB = 4


def _detile_body(tt_hbm, out_hbm, *rest):
    in_b = rest[0:DNB]
    rm_b = rest[DNB:2 * DNB]
    s_in = rest[2 * DNB:3 * DNB]
    s_out = rest[3 * DNB:4 * DNB]

    wid = lax.axis_index("s") * NC + lax.axis_index("c")
    start = wid * BLK_BASE + jnp.minimum(wid, BLK_REM)
    count = BLK_BASE + (wid < BLK_REM).astype(jnp.int32)
    lanes = lax.iota(jnp.int32, 16)

    def issue_in(blk, p):
        n0 = blk * TB
        for k in range(4):
            pltpu.make_async_copy(
                tt_hbm.at[pl.ds(8 * k, 8), pl.ds(n0, TB)],
                in_b[p].at[pl.ds(8 * k, 8), pl.ds(0, TB)], s_in[p]).start()

    def wait_in(p):
        for k in range(4):
            pltpu.make_async_copy(
                tt_hbm.at[pl.ds(0, 8), pl.ds(0, TB)],
                in_b[p].at[pl.ds(8 * k, 8), pl.ds(0, TB)], s_in[p]).wait()

    def out_desc(blk, p):
        return pltpu.make_async_copy(
            rm_b[p], out_hbm.at[pl.ds(blk * TB * D, TB * D)], s_out[p])

    for p in range(DNB):
        @pl.when(p < count)
        def _prime():
            issue_in(start + p, p)

    def blk_fn(i, carry):
        for p in range(DNB):
            idx = i * DNB + p

            @pl.when(idx < count)
            def _do():
                blk = start + idx
                wait_in(p)

                @pl.when(idx >= DNB)
                def _wait_out():
                    out_desc(blk, p).wait()

                def g_fn(g, carry2):
                    perm = (lanes + g) & 15
                    for f0 in range(0, D, 16):
                        for bt in range(TB // 16):
                            tvec = bt * 16 + lanes
                            fvec = f0 + perm
                            v = plsc.load_gather(in_b[p], [fvec, tvec])
                            plsc.store_scatter(
                                rm_b[p], [tvec * D + fvec], v * SCALE)
                    return carry2

                lax.fori_loop(0, 16, g_fn, 0, unroll=2)

                out_desc(blk, p).start()

                @pl.when(idx + DNB < count)
                def _refill():
                    issue_in(blk + DNB, p)

        return carry

    lax.fori_loop(0, (BLK_BASE + DNB) // DNB, blk_fn, 0)

    for p in range(DNB):
        @pl.when(count >= DNB - p)
        def _drain():
            out_desc(start, p).wait()


def _body(idx_hbm, table_hbm, out_hbm, idx_v, *rest):
    rows_b = rest[0:CPS]
    trans_b = rest[CPS:2 * CPS]
    sem_in = rest[2 * CPS:3 * CPS]
    sem_out = rest[3 * CPS:4 * CPS]

    wid = lax.axis_index("s") * NC + lax.axis_index("c")
    n_base = wid * N_PER_W
    pltpu.sync_copy(idx_hbm.at[wid], idx_v)

    lanes = lax.iota(jnp.int32, 16)

    for c in range(CPS):
        pltpu.make_async_copy(
            table_hbm.at[idx_v.at[c]], rows_b[c], sem_in[c]).start()

    def slot_fn(s, carry):
        for c in range(CPS):
            j = s * CPS + c
            pltpu.make_async_copy(
                table_hbm.at[idx_v.at[j]], rows_b[c], sem_in[c]).wait()
            tcol = (n_base + c * C) // C
            out_desc = [
                pltpu.make_async_copy(
                    trans_b[c].at[pl.ds(8 * k, 8), pl.ds(0, C)],
                    out_hbm.at[4 * s + k, tcol], sem_out[c])
                for k in range(4)]

            @pl.when(s > 0)
            def _wait_prev():
                for k in range(4):
                    out_desc[k].wait()

            def row_fn(r, carry2):
                rcol = jnp.full((16,), 0, jnp.int32) + r
                for f0 in range(0, D, 16):
                    v = rows_b[c][r, pl.ds(f0, 16)]
                    plsc.store_scatter(trans_b[c], [lanes + f0, rcol], v)
                return carry2

            lax.fori_loop(0, C, row_fn, 0, unroll=8)

            @pl.when(s + 1 < N_SLOT)
            def _refill():
                jn = jnp.minimum(j + CPS, CHUNKS - 1)
                pltpu.make_async_copy(
                    table_hbm.at[idx_v.at[jn]], rows_b[c], sem_in[c]).start()

            for k in range(4):
                out_desc[k].start()
        return carry

    lax.fori_loop(0, N_SLOT, slot_fn, 0)

    for c in range(CPS):
        for k in range(4):
            pltpu.make_async_copy(
                trans_b[c].at[pl.ds(8 * k, 8), pl.ds(0, C)],
                out_hbm.at[4 * (N_SLOT - 1) + k, (n_base + c * C) // C],
                sem_out[c]).wait()


@functools.partial(jax.jit, static_argnums=())
def _detile(tt):
    mesh = plsc.VectorSubcoreMesh(core_axis_name="c", subcore_axis_name="s")
    scratch = [pltpu.VMEM((D, 2 * TB), jnp.float32) for _ in range(DNB)]
    scratch += [pltpu.VMEM((TB * D,), jnp.float32) for _ in range(DNB)]
    scratch += [pltpu.SemaphoreType.DMA for _ in range(2 * DNB)]
    k = pl.kernel(
        _detile_body,
        out_type=jax.ShapeDtypeStruct((VOCAB * D,), jnp.float32),
        mesh=mesh,
        scratch_types=scratch,
        compiler_params=pltpu.CompilerParams(
            use_tc_tiling_on_sc=True, needs_layout_passes=False),
    )
    return k(tt)


@functools.partial(jax.jit, static_argnums=())
def _lookup(idx, table):
    mesh = plsc.VectorSubcoreMesh(core_axis_name="c", subcore_axis_name="s")
    scratch = [pltpu.VMEM((CHUNKS, C), jnp.int32)]
    scratch += [pltpu.VMEM((C, D), jnp.float32) for _ in range(CPS)]
    scratch += [pltpu.VMEM((D, C + 1), jnp.float32) for _ in range(CPS)]
    scratch += [pltpu.SemaphoreType.DMA for _ in range(2 * CPS)]
    k = pl.kernel(
        _body,
        out_type=jax.ShapeDtypeStruct(
            (N_SLOT * D // 8, N_BATCH // C, 8, C), jnp.float32),
        mesh=mesh,
        scratch_types=scratch,
        compiler_params=pltpu.CompilerParams(
            use_tc_tiling_on_sc=False, needs_layout_passes=False),
    )
    return k(idx, table)


def kernel(inputs, embeddings):
    idx = (inputs.astype(jnp.int32).T
           .reshape(N_SLOT, NW, CPS, C)
           .transpose(1, 0, 2, 3)
           .reshape(NW, CHUNKS, C))
    table = _detile(embeddings.T).reshape(VOCAB, D)
    out = _lookup(idx, table)
    out = out.transpose(0, 2, 1, 3).reshape(N_SLOT, D, N_BATCH)
    return jnp.transpose(out, (2, 0, 1))
